# Initial kernel scaffold; baseline (speedup 1.0000x reference)
import jax
import jax.numpy as jnp
from jax.experimental import pallas as pl


def kernel(all_embeddings, assoc, src, pos_dst, neg_dst_surv, neg_src_surv, neg_dst, last_update, cur_time, et, W0, b0, W1, b1, psi, alpha, w_t):
    raise NotImplementedError("write your pallas kernel here")


# speedup vs baseline: 7.3492x; 7.3492x over previous
import functools

import jax
import jax.numpy as jnp
from jax import lax
from jax.experimental import pallas as pl
from jax.experimental.pallas import tpu as pltpu
from jax.experimental.pallas import tpu_sc as plsc

N = 100000
B = 4096
S = 20
D = 32
TRAIN_TD_MAX = 1.0

NC = 2
NS = 16
NW = NC * NS
L = 16
EPW = B // NW
MPW = EPW * S
ROWS = N // 4
PROJ_BLK = 1000


def _proj_body(emb_ref, w_ref, o0_ref, o1_ref):
    o = lax.dot_general(emb_ref[...], w_ref[...], (((1,), (0,)), ((), ())),
                        preferred_element_type=jnp.float32)
    o0_ref[...] = o[:, :4]
    o1_ref[...] = o[:, 4:]


_proj_call = pl.pallas_call(
    _proj_body,
    grid=(ROWS // PROJ_BLK,),
    in_specs=[
        pl.BlockSpec((PROJ_BLK, 4 * D), lambda i: (i, 0)),
        pl.BlockSpec((4 * D, 8), lambda i: (0, 0)),
    ],
    out_specs=[
        pl.BlockSpec((PROJ_BLK, 4), lambda i: (i, 0)),
        pl.BlockSpec((PROJ_BLK, 4), lambda i: (i, 0)),
    ],
    out_shape=[
        jax.ShapeDtypeStruct((ROWS, 4), jnp.float32),
        jax.ShapeDtypeStruct((ROWS, 4), jnp.float32),
    ],
)


def _softplus(x):
    t = jnp.exp(-jnp.abs(x))
    z = t / (2.0 + t)
    z2 = z * z
    l1p = 2.0 * z * (1.0 + z2 * (1.0 / 3.0 + z2 * (1.0 / 5.0 + z2 * (1.0 / 7.0 + z2 * (1.0 / 9.0)))))
    return jnp.maximum(x, 0.0) + l1p


def _hawkes(ssum, psi_, al_, wt_, b_, td):
    g = 0.5 * ssum + b_ + al_ * jnp.exp(-wt_ * (td / TRAIN_TD_MAX))
    x = jnp.clip(g / (psi_ + 1e-7), -75.0, 75.0)
    return psi_ * _softplus(x)


def _sc_body(assoc_h, s0_h, s1_h, lu_h, src_h, dst_h, neg_h, et_h, ct_h,
             nds_h, nss_h, par_h,
             lam_o, lamn_o, susum_o, svsum_o,
             src_v, dst_v, neg_v, et_v, ct_v, nds_v, nss_v,
             a_src, a_dst, a_neg, a_nds, a_nss,
             s0u, s1u, luu, s0v, s1v, luv, s0w, s1w, luw,
             s0k, s1k, luk, s0m, s1m, lum,
             par_v, lam_v, lamn_v, susum_v, svsum_v, sem1, sem2):
    wid = lax.axis_index("s") * NC + lax.axis_index("c")
    eb = wid * EPW

    pltpu.sync_copy(src_h.at[pl.ds(eb, EPW)], src_v)
    pltpu.sync_copy(dst_h.at[pl.ds(eb, EPW)], dst_v)
    pltpu.sync_copy(neg_h.at[pl.ds(eb, EPW)], neg_v)
    pltpu.sync_copy(et_h.at[pl.ds(eb, EPW)], et_v)
    pltpu.sync_copy(ct_h.at[pl.ds(eb, EPW)], ct_v)
    pltpu.sync_copy(nds_h.at[wid], nds_v)
    pltpu.sync_copy(nss_h.at[wid], nss_v)
    pltpu.sync_copy(par_h, par_v)

    hop1 = [
        pltpu.async_copy(assoc_h.at[src_v], a_src, sem1),
        pltpu.async_copy(assoc_h.at[dst_v], a_dst, sem1),
        pltpu.async_copy(assoc_h.at[neg_v], a_neg, sem1),
    ]
    for r in range(S):
        hop1.append(pltpu.async_copy(assoc_h.at[nds_v.at[r]], a_nds.at[r], sem1))
        hop1.append(pltpu.async_copy(assoc_h.at[nss_v.at[r]], a_nss.at[r], sem1))
    for c in hop1:
        c.wait()

    hop2 = []
    for idx, outs in ((a_src, (s0u, s1u, luu)), (a_dst, (s0v, s1v, luv)),
                      (a_neg, (s0w, s1w, luw))):
        for tab, ov in zip((s0_h, s1_h, lu_h), outs):
            hop2.append(pltpu.async_copy(tab.at[idx], ov, sem2))
    for idx, outs in ((a_nds, (s0k, s1k, luk)), (a_nss, (s0m, s1m, lum))):
        for tab, ov in zip((s0_h, s1_h, lu_h), outs):
            for r in range(S):
                hop2.append(pltpu.async_copy(tab.at[idx.at[r]], ov.at[r], sem2))
    for c in hop2:
        c.wait()

    psi0 = par_v[pl.ds(0 * L, L)]
    psi1 = par_v[pl.ds(1 * L, L)]
    al0 = par_v[pl.ds(2 * L, L)]
    al1 = par_v[pl.ds(3 * L, L)]
    wt0 = par_v[pl.ds(4 * L, L)]
    wt1 = par_v[pl.ds(5 * L, L)]
    b0v = par_v[pl.ds(6 * L, L)]
    b1v = par_v[pl.ds(7 * L, L)]

    def chunk_body(c, carry):
        sl = pl.ds(c * L, L)
        etm = et_v[sl] > 0
        ctc = ct_v[sl]
        lts = luu[sl]
        ltd = luv[sl]
        s0uc = s0u[sl]
        s1uc = s1u[sl]
        s0dc = s0v[sl]
        s1dc = s1v[sl]

        psie = jnp.where(etm, psi1, psi0)
        ale = jnp.where(etm, al1, al0)
        wte = jnp.where(etm, wt1, wt0)
        be = jnp.where(etm, b1v, b0v)

        ssum = jnp.where(etm, s1uc + s1dc, s0uc + s0dc)
        lam_v[sl] = _hawkes(ssum, psie, ale, wte, be,
                            ctc - jnp.maximum(lts, ltd))

        ssumn = jnp.where(etm, s1uc + s1w[sl], s0uc + s0w[sl])
        lamn_v[sl] = _hawkes(ssumn, psie, ale, wte, be,
                             ctc - jnp.maximum(lts, luw[sl]))

        def jbody(j, accs):
            au, av = accs
            ksl = pl.ds(c * L, L)
            s0kj = s0k[j, ksl]
            s1kj = s1k[j, ksl]
            lukj = luk[j, ksl]
            tdu = ctc - jnp.maximum(lts, lukj)
            au = (au
                  + _hawkes(s0uc + s0kj, psi0, al0, wt0, b0v, tdu)
                  + _hawkes(s1uc + s1kj, psi1, al1, wt1, b1v, tdu))
            s0mj = s0m[j, ksl]
            s1mj = s1m[j, ksl]
            lumj = lum[j, ksl]
            tdv = ctc - jnp.maximum(lumj, ltd)
            av = (av
                  + _hawkes(s0mj + s0dc, psi0, al0, wt0, b0v, tdv)
                  + _hawkes(s1mj + s1dc, psi1, al1, wt1, b1v, tdv))
            return (au, av)

        zero = jnp.zeros((L,), jnp.float32)
        acc_u, acc_v = lax.fori_loop(0, S, jbody, (zero, zero))
        susum_v[sl] = acc_u
        svsum_v[sl] = acc_v
        return carry

    lax.fori_loop(0, EPW // L, chunk_body, 0)

    pltpu.sync_copy(lam_v, lam_o.at[pl.ds(eb, EPW)])
    pltpu.sync_copy(lamn_v, lamn_o.at[pl.ds(eb, EPW)])
    pltpu.sync_copy(susum_v, susum_o.at[pl.ds(eb, EPW)])
    pltpu.sync_copy(svsum_v, svsum_o.at[pl.ds(eb, EPW)])


_sc_call = pl.kernel(
    _sc_body,
    out_type=[jax.ShapeDtypeStruct((B,), jnp.float32)] * 4,
    mesh=plsc.VectorSubcoreMesh(core_axis_name="c", subcore_axis_name="s",
                                num_cores=NC, num_subcores=NS),
    scratch_types=[
        pltpu.VMEM((EPW,), jnp.int32),
        pltpu.VMEM((EPW,), jnp.int32),
        pltpu.VMEM((EPW,), jnp.int32),
        pltpu.VMEM((EPW,), jnp.int32),
        pltpu.VMEM((EPW,), jnp.float32),
        pltpu.VMEM((S, EPW), jnp.int32),
        pltpu.VMEM((S, EPW), jnp.int32),
        pltpu.VMEM((EPW,), jnp.int32),
        pltpu.VMEM((EPW,), jnp.int32),
        pltpu.VMEM((EPW,), jnp.int32),
        pltpu.VMEM((S, EPW), jnp.int32),
        pltpu.VMEM((S, EPW), jnp.int32),
        pltpu.VMEM((EPW,), jnp.float32),
        pltpu.VMEM((EPW,), jnp.float32),
        pltpu.VMEM((EPW,), jnp.float32),
        pltpu.VMEM((EPW,), jnp.float32),
        pltpu.VMEM((EPW,), jnp.float32),
        pltpu.VMEM((EPW,), jnp.float32),
        pltpu.VMEM((EPW,), jnp.float32),
        pltpu.VMEM((EPW,), jnp.float32),
        pltpu.VMEM((EPW,), jnp.float32),
        pltpu.VMEM((S, EPW), jnp.float32),
        pltpu.VMEM((S, EPW), jnp.float32),
        pltpu.VMEM((S, EPW), jnp.float32),
        pltpu.VMEM((S, EPW), jnp.float32),
        pltpu.VMEM((S, EPW), jnp.float32),
        pltpu.VMEM((S, EPW), jnp.float32),
        pltpu.VMEM((8 * L,), jnp.float32),
        pltpu.VMEM((EPW,), jnp.float32),
        pltpu.VMEM((EPW,), jnp.float32),
        pltpu.VMEM((EPW,), jnp.float32),
        pltpu.VMEM((EPW,), jnp.float32),
        pltpu.SemaphoreType.DMA,
        pltpu.SemaphoreType.DMA,
    ],
)


def _fin_body(lam_ref, lamn_ref, su_ref, sv_ref,
              ll_ref, lsu_ref, lsv_ref, cp_ref, cn_ref):
    lam = lam_ref[...]
    lamn = lamn_ref[...]
    su = su_ref[...]
    sv = sv_ref[...]
    ll_ref[...] = -jnp.sum(jnp.log(lam + 1e-7), keepdims=True) / B
    lsu_ref[...] = jnp.sum(su, keepdims=True) / (S * B)
    lsv_ref[...] = jnp.sum(sv, keepdims=True) / (S * B)
    surv = jnp.exp(-(su + sv) / S)
    cp_ref[...] = lam * surv
    cn_ref[...] = lamn * surv


_fin_call = pl.pallas_call(
    _fin_body,
    out_shape=[
        jax.ShapeDtypeStruct((1, 1), jnp.float32),
        jax.ShapeDtypeStruct((1, 1), jnp.float32),
        jax.ShapeDtypeStruct((1, 1), jnp.float32),
        jax.ShapeDtypeStruct((B // 128, 128), jnp.float32),
        jax.ShapeDtypeStruct((B // 128, 128), jnp.float32),
    ],
)


def kernel(all_embeddings, assoc, src, pos_dst, neg_dst_surv, neg_src_surv,
           neg_dst, last_update, cur_time, et, W0, b0, W1, b1, psi, alpha, w_t):
    ws0 = (W0[:D] + W0[D:]).astype(jnp.float32)
    ws1 = (W1[:D] + W1[D:]).astype(jnp.float32)
    eye4 = jnp.eye(4, dtype=jnp.float32)
    wmat = jnp.concatenate(
        [jnp.kron(eye4, ws0[:, None]), jnp.kron(eye4, ws1[:, None])], axis=1)
    s0m, s1m = _proj_call(all_embeddings.reshape(ROWS, 4 * D), wmat)
    s0 = s0m.reshape(N)
    s1 = s1m.reshape(N)

    par = jnp.concatenate([
        jnp.broadcast_to(p.astype(jnp.float32), (L,))
        for p in (psi[0], psi[1], alpha[0], alpha[1], w_t[0], w_t[1],
                  b0[0], b1[0])
    ])

    nds_t = (neg_dst_surv.astype(jnp.int32)
             .reshape(NW, EPW, S).transpose(0, 2, 1))
    nss_t = (neg_src_surv.astype(jnp.int32)
             .reshape(NW, EPW, S).transpose(0, 2, 1))

    lam, lamn, susum, svsum = _sc_call(
        assoc.astype(jnp.int32), s0, s1, last_update,
        src.astype(jnp.int32), pos_dst.astype(jnp.int32),
        neg_dst.astype(jnp.int32), et.astype(jnp.int32), cur_time,
        nds_t, nss_t, par)

    ll, lsu, lsv, cp, cn = _fin_call(
        lam.reshape(B // 128, 128), lamn.reshape(B // 128, 128),
        susum.reshape(B // 128, 128), svsum.reshape(B // 128, 128))
    return (ll[0, 0], lsu[0, 0], lsv[0, 0], cp.reshape(B), cn.reshape(B))


# speedup vs baseline: 9.3215x; 1.2684x over previous
import functools

import jax
import jax.numpy as jnp
from jax import lax
from jax.experimental import pallas as pl
from jax.experimental.pallas import tpu as pltpu
from jax.experimental.pallas import tpu_sc as plsc

N = 100000
B = 4096
S = 20
D = 32
TRAIN_TD_MAX = 1.0

NC = 2
NS = 16
NW = NC * NS
L = 16
EPW = B // NW
ECH = EPW // L
ROWS = N // 4
PROJ_BLK = 5000


def _proj_body(emb_ref, w_ref, o_ref):
    o_ref[...] = lax.dot_general(emb_ref[...], w_ref[...],
                                 (((1,), (0,)), ((), ())),
                                 preferred_element_type=jnp.float32)


_proj_call = pl.pallas_call(
    _proj_body,
    grid=(ROWS // PROJ_BLK,),
    in_specs=[
        pl.BlockSpec((PROJ_BLK, 4 * D), lambda i: (i, 0)),
        pl.BlockSpec((4 * D, 4 * D), lambda i: (0, 0)),
    ],
    out_specs=pl.BlockSpec((PROJ_BLK, 4 * D), lambda i: (i, 0)),
    out_shape=jax.ShapeDtypeStruct((ROWS, 4 * D), jnp.float32),
)


def _softplus(x):
    t = jnp.exp(-jnp.abs(x))
    z = t / (2.0 + t)
    z2 = z * z
    l1p = 2.0 * z * (1.0 + z2 * (1.0 / 3.0 + z2 * (1.0 / 5.0 + z2 * (1.0 / 7.0 + z2 * (1.0 / 9.0)))))
    return jnp.maximum(x, 0.0) + l1p


def _hawkes(ssum, psi_, al_, wt_, b_, td):
    g = 0.5 * ssum + b_ + al_ * jnp.exp(-wt_ * (td / TRAIN_TD_MAX))
    x = jnp.clip(g / (psi_ + 1e-7), -75.0, 75.0)
    return psi_ * _softplus(x)


def _tab_idx(a):
    return (a >> 2) * 128 + (a & 3) * 2


def _sc_body(stab_h, assoc_h, lu_h, src_h, dst_h, neg_h, et_h, ct_h,
             nds_h, nss_h, par_h,
             lam_o, lamn_o, susum_o, svsum_o,
             src_v, dst_v, neg_v, et_v, ct_v, rix, nds_v, nss_v,
             a_src, a_dst, a_neg, a_nds, a_nss,
             i0s, i1s, i0d, i1d, i0n, i1n,
             i0k, i1k, i0m, i1m,
             s0u, s1u, luu, s0v, s1v, luv, s0w, s1w, luw,
             s0k, s1k, luk, s0m, s1m, lum,
             par_v, lam_v, lamn_v, susum_v, svsum_v,
             sem0, sem1, sem2):
    wid = lax.axis_index("s") * NC + lax.axis_index("c")
    eb = wid * EPW
    mb = wid * EPW * S

    pltpu.sync_copy(src_h.at[pl.ds(eb, EPW)], src_v)
    pltpu.sync_copy(dst_h.at[pl.ds(eb, EPW)], dst_v)
    pltpu.sync_copy(neg_h.at[pl.ds(eb, EPW)], neg_v)
    pltpu.sync_copy(et_h.at[pl.ds(eb, EPW)], et_v)
    pltpu.sync_copy(ct_h.at[pl.ds(eb, EPW)], ct_v)
    pltpu.sync_copy(par_h, par_v)

    lanes = lax.iota(jnp.int32, L)
    for r in range(S):
        for t in range(ECH):
            rix[r, pl.ds(t * L, L)] = mb + (lanes + t * L) * S + r
    hop0 = []
    for r in range(S):
        hop0.append(pltpu.async_copy(nds_h.at[rix.at[r]], nds_v.at[r], sem0))
        hop0.append(pltpu.async_copy(nss_h.at[rix.at[r]], nss_v.at[r], sem0))
    for c in hop0:
        c.wait()

    hop1 = [
        pltpu.async_copy(assoc_h.at[src_v], a_src, sem1),
        pltpu.async_copy(assoc_h.at[dst_v], a_dst, sem1),
        pltpu.async_copy(assoc_h.at[neg_v], a_neg, sem1),
    ]
    for r in range(S):
        hop1.append(pltpu.async_copy(assoc_h.at[nds_v.at[r]], a_nds.at[r], sem1))
        hop1.append(pltpu.async_copy(assoc_h.at[nss_v.at[r]], a_nss.at[r], sem1))
    for c in hop1:
        c.wait()

    for t in range(ECH):
        sl = pl.ds(t * L, L)
        for a_ref, i0_ref, i1_ref in ((a_src, i0s, i1s), (a_dst, i0d, i1d),
                                      (a_neg, i0n, i1n)):
            i0 = _tab_idx(a_ref[sl])
            i0_ref[sl] = i0
            i1_ref[sl] = i0 + 1

    def idx_body(r, carry):
        for t in range(ECH):
            sl = pl.ds(t * L, L)
            for a_ref, i0_ref, i1_ref in ((a_nds, i0k, i1k), (a_nss, i0m, i1m)):
                i0 = _tab_idx(a_ref[r, sl])
                i0_ref[r, sl] = i0
                i1_ref[r, sl] = i0 + 1
        return carry

    lax.fori_loop(0, S, idx_body, 0)

    hop2 = []
    for a_ref, i0_ref, i1_ref, outs in (
            (a_src, i0s, i1s, (s0u, s1u, luu)),
            (a_dst, i0d, i1d, (s0v, s1v, luv)),
            (a_neg, i0n, i1n, (s0w, s1w, luw))):
        hop2.append(pltpu.async_copy(stab_h.at[i0_ref], outs[0], sem2))
        hop2.append(pltpu.async_copy(stab_h.at[i1_ref], outs[1], sem2))
        hop2.append(pltpu.async_copy(lu_h.at[a_ref], outs[2], sem2))
    for a_ref, i0_ref, i1_ref, outs in (
            (a_nds, i0k, i1k, (s0k, s1k, luk)),
            (a_nss, i0m, i1m, (s0m, s1m, lum))):
        for r in range(S):
            hop2.append(pltpu.async_copy(stab_h.at[i0_ref.at[r]], outs[0].at[r], sem2))
            hop2.append(pltpu.async_copy(stab_h.at[i1_ref.at[r]], outs[1].at[r], sem2))
            hop2.append(pltpu.async_copy(lu_h.at[a_ref.at[r]], outs[2].at[r], sem2))
    for c in hop2:
        c.wait()

    psi0 = par_v[pl.ds(0 * L, L)]
    psi1 = par_v[pl.ds(1 * L, L)]
    al0 = par_v[pl.ds(2 * L, L)]
    al1 = par_v[pl.ds(3 * L, L)]
    wt0 = par_v[pl.ds(4 * L, L)]
    wt1 = par_v[pl.ds(5 * L, L)]
    b0v = par_v[pl.ds(6 * L, L)]
    b1v = par_v[pl.ds(7 * L, L)]

    def chunk_body(c, carry):
        sl = pl.ds(c * L, L)
        etm = et_v[sl] > 0
        ctc = ct_v[sl]
        lts = luu[sl]
        ltd = luv[sl]
        s0uc = s0u[sl]
        s1uc = s1u[sl]
        s0dc = s0v[sl]
        s1dc = s1v[sl]

        psie = jnp.where(etm, psi1, psi0)
        ale = jnp.where(etm, al1, al0)
        wte = jnp.where(etm, wt1, wt0)
        be = jnp.where(etm, b1v, b0v)

        ssum = jnp.where(etm, s1uc + s1dc, s0uc + s0dc)
        lam_v[sl] = _hawkes(ssum, psie, ale, wte, be,
                            ctc - jnp.maximum(lts, ltd))

        ssumn = jnp.where(etm, s1uc + s1w[sl], s0uc + s0w[sl])
        lamn_v[sl] = _hawkes(ssumn, psie, ale, wte, be,
                             ctc - jnp.maximum(lts, luw[sl]))

        def jbody(j, accs):
            au, av = accs
            ksl = pl.ds(c * L, L)
            tdu = ctc - jnp.maximum(lts, luk[j, ksl])
            au = (au
                  + _hawkes(s0uc + s0k[j, ksl], psi0, al0, wt0, b0v, tdu)
                  + _hawkes(s1uc + s1k[j, ksl], psi1, al1, wt1, b1v, tdu))
            tdv = ctc - jnp.maximum(lum[j, ksl], ltd)
            av = (av
                  + _hawkes(s0m[j, ksl] + s0dc, psi0, al0, wt0, b0v, tdv)
                  + _hawkes(s1m[j, ksl] + s1dc, psi1, al1, wt1, b1v, tdv))
            return (au, av)

        zero = jnp.zeros((L,), jnp.float32)
        acc_u, acc_v = lax.fori_loop(0, S, jbody, (zero, zero))
        susum_v[sl] = acc_u
        svsum_v[sl] = acc_v
        return carry

    lax.fori_loop(0, ECH, chunk_body, 0)

    pltpu.sync_copy(lam_v, lam_o.at[pl.ds(eb, EPW)])
    pltpu.sync_copy(lamn_v, lamn_o.at[pl.ds(eb, EPW)])
    pltpu.sync_copy(susum_v, susum_o.at[pl.ds(eb, EPW)])
    pltpu.sync_copy(svsum_v, svsum_o.at[pl.ds(eb, EPW)])


_sc_call = pl.kernel(
    _sc_body,
    out_type=[jax.ShapeDtypeStruct((B,), jnp.float32)] * 4,
    mesh=plsc.VectorSubcoreMesh(core_axis_name="c", subcore_axis_name="s",
                                num_cores=NC, num_subcores=NS),
    scratch_types=[
        pltpu.VMEM((EPW,), jnp.int32),
        pltpu.VMEM((EPW,), jnp.int32),
        pltpu.VMEM((EPW,), jnp.int32),
        pltpu.VMEM((EPW,), jnp.int32),
        pltpu.VMEM((EPW,), jnp.float32),
        pltpu.VMEM((S, EPW), jnp.int32),
        pltpu.VMEM((S, EPW), jnp.int32),
        pltpu.VMEM((S, EPW), jnp.int32),
        pltpu.VMEM((EPW,), jnp.int32),
        pltpu.VMEM((EPW,), jnp.int32),
        pltpu.VMEM((EPW,), jnp.int32),
        pltpu.VMEM((S, EPW), jnp.int32),
        pltpu.VMEM((S, EPW), jnp.int32),
        pltpu.VMEM((EPW,), jnp.int32),
        pltpu.VMEM((EPW,), jnp.int32),
        pltpu.VMEM((EPW,), jnp.int32),
        pltpu.VMEM((EPW,), jnp.int32),
        pltpu.VMEM((EPW,), jnp.int32),
        pltpu.VMEM((EPW,), jnp.int32),
        pltpu.VMEM((S, EPW), jnp.int32),
        pltpu.VMEM((S, EPW), jnp.int32),
        pltpu.VMEM((S, EPW), jnp.int32),
        pltpu.VMEM((S, EPW), jnp.int32),
        pltpu.VMEM((EPW,), jnp.float32),
        pltpu.VMEM((EPW,), jnp.float32),
        pltpu.VMEM((EPW,), jnp.float32),
        pltpu.VMEM((EPW,), jnp.float32),
        pltpu.VMEM((EPW,), jnp.float32),
        pltpu.VMEM((EPW,), jnp.float32),
        pltpu.VMEM((EPW,), jnp.float32),
        pltpu.VMEM((EPW,), jnp.float32),
        pltpu.VMEM((EPW,), jnp.float32),
        pltpu.VMEM((S, EPW), jnp.float32),
        pltpu.VMEM((S, EPW), jnp.float32),
        pltpu.VMEM((S, EPW), jnp.float32),
        pltpu.VMEM((S, EPW), jnp.float32),
        pltpu.VMEM((S, EPW), jnp.float32),
        pltpu.VMEM((S, EPW), jnp.float32),
        pltpu.VMEM((8 * L,), jnp.float32),
        pltpu.VMEM((EPW,), jnp.float32),
        pltpu.VMEM((EPW,), jnp.float32),
        pltpu.VMEM((EPW,), jnp.float32),
        pltpu.VMEM((EPW,), jnp.float32),
        pltpu.SemaphoreType.DMA,
        pltpu.SemaphoreType.DMA,
        pltpu.SemaphoreType.DMA,
    ],
)


def _fin_body(lam_ref, lamn_ref, su_ref, sv_ref,
              ll_ref, lsu_ref, lsv_ref, cp_ref, cn_ref):
    lam = lam_ref[...]
    lamn = lamn_ref[...]
    su = su_ref[...]
    sv = sv_ref[...]
    ll_ref[...] = -jnp.sum(jnp.log(lam + 1e-7), keepdims=True) / B
    lsu_ref[...] = jnp.sum(su, keepdims=True) / (S * B)
    lsv_ref[...] = jnp.sum(sv, keepdims=True) / (S * B)
    surv = jnp.exp(-(su + sv) / S)
    cp_ref[...] = lam * surv
    cn_ref[...] = lamn * surv


_fin_call = pl.pallas_call(
    _fin_body,
    out_shape=[
        jax.ShapeDtypeStruct((1, 1), jnp.float32),
        jax.ShapeDtypeStruct((1, 1), jnp.float32),
        jax.ShapeDtypeStruct((1, 1), jnp.float32),
        jax.ShapeDtypeStruct((B // 128, 128), jnp.float32),
        jax.ShapeDtypeStruct((B // 128, 128), jnp.float32),
    ],
)


def kernel(all_embeddings, assoc, src, pos_dst, neg_dst_surv, neg_src_surv,
           neg_dst, last_update, cur_time, et, W0, b0, W1, b1, psi, alpha, w_t):
    ws0 = (W0[:D] + W0[D:]).astype(jnp.float32)
    ws1 = (W1[:D] + W1[D:]).astype(jnp.float32)
    base = jnp.stack([ws0, ws1], axis=1)
    wbig = jnp.pad(jnp.kron(jnp.eye(4, dtype=jnp.float32), base),
                   ((0, 0), (0, 4 * D - 8)))
    stab = _proj_call(all_embeddings.reshape(ROWS, 4 * D), wbig)

    par = jnp.concatenate([
        jnp.broadcast_to(p.astype(jnp.float32), (L,))
        for p in (psi[0], psi[1], alpha[0], alpha[1], w_t[0], w_t[1],
                  b0[0], b1[0])
    ])

    lam, lamn, susum, svsum = _sc_call(
        stab.reshape(ROWS * 4 * D), assoc.astype(jnp.int32), last_update,
        src.astype(jnp.int32), pos_dst.astype(jnp.int32),
        neg_dst.astype(jnp.int32), et.astype(jnp.int32), cur_time,
        neg_dst_surv.astype(jnp.int32), neg_src_surv.astype(jnp.int32), par)

    ll, lsu, lsv, cp, cn = _fin_call(
        lam.reshape(B // 128, 128), lamn.reshape(B // 128, 128),
        susum.reshape(B // 128, 128), svsum.reshape(B // 128, 128))
    return (ll[0, 0], lsu[0, 0], lsv[0, 0], cp.reshape(B), cn.reshape(B))


# speedup vs baseline: 9.4112x; 1.0096x over previous
import functools

import jax
import jax.numpy as jnp
from jax import lax
from jax.experimental import pallas as pl
from jax.experimental.pallas import tpu as pltpu
from jax.experimental.pallas import tpu_sc as plsc

N = 100000
B = 4096
S = 20
D = 32
TRAIN_TD_MAX = 1.0

NC = 2
NS = 16
NW = NC * NS
L = 16
EPW = B // NW
ECH = EPW // L
NPAD = 100096
PROJ_BLK = 50048


def _proj_body(emb_ref, w_ref, o_ref):
    o_ref[...] = lax.dot_general(w_ref[...], emb_ref[...],
                                 (((1,), (1,)), ((), ())),
                                 preferred_element_type=jnp.float32)


_proj_call = pl.pallas_call(
    _proj_body,
    grid=(NPAD // PROJ_BLK,),
    in_specs=[
        pl.BlockSpec((PROJ_BLK, D), lambda i: (i, 0)),
        pl.BlockSpec((8, D), lambda i: (0, 0)),
    ],
    out_specs=pl.BlockSpec((8, PROJ_BLK), lambda i: (0, i)),
    out_shape=jax.ShapeDtypeStruct((8, NPAD), jnp.float32),
)


def _softplus(x):
    t = jnp.exp(-jnp.abs(x))
    z = t / (2.0 + t)
    z2 = z * z
    l1p = 2.0 * z * (1.0 + z2 * (1.0 / 3.0 + z2 * (1.0 / 5.0 + z2 * (1.0 / 7.0 + z2 * (1.0 / 9.0)))))
    return jnp.maximum(x, 0.0) + l1p


def _hawkes(ssum, psi_, al_, wt_, b_, td):
    g = 0.5 * ssum + b_ + al_ * jnp.exp(-wt_ * (td / TRAIN_TD_MAX))
    x = jnp.clip(g / (psi_ + 1e-7), -75.0, 75.0)
    return psi_ * _softplus(x)


def _sc_body(stab_h, assoc_h, lu_h, src_h, dst_h, neg_h, et_h, ct_h,
             nds_h, nss_h, par_h,
             lam_o, lamn_o, susum_o, svsum_o,
             src_v, dst_v, neg_v, et_v, ct_v, rix, nds_v, nss_v,
             a_src, a_dst, a_neg, a_nds, a_nss,
             i1s, i1d, i1n, i1k, i1m,
             s0u, s1u, luu, s0v, s1v, luv, s0w, s1w, luw,
             s0k, s1k, luk, s0m, s1m, lum,
             par_v, lam_v, lamn_v, susum_v, svsum_v,
             sem0, sem1, sem2):
    wid = lax.axis_index("s") * NC + lax.axis_index("c")
    eb = wid * EPW
    mb = wid * EPW * S

    pltpu.sync_copy(src_h.at[pl.ds(eb, EPW)], src_v)
    pltpu.sync_copy(dst_h.at[pl.ds(eb, EPW)], dst_v)
    pltpu.sync_copy(neg_h.at[pl.ds(eb, EPW)], neg_v)
    pltpu.sync_copy(et_h.at[pl.ds(eb, EPW)], et_v)
    pltpu.sync_copy(ct_h.at[pl.ds(eb, EPW)], ct_v)
    pltpu.sync_copy(par_h, par_v)

    lanes = lax.iota(jnp.int32, L)
    for r in range(S):
        for t in range(ECH):
            rix[r, pl.ds(t * L, L)] = mb + (lanes + t * L) * S + r
    hop0 = []
    for r in range(S):
        hop0.append(pltpu.async_copy(nds_h.at[rix.at[r]], nds_v.at[r], sem0))
        hop0.append(pltpu.async_copy(nss_h.at[rix.at[r]], nss_v.at[r], sem0))
    for c in hop0:
        c.wait()

    hop1 = [
        pltpu.async_copy(assoc_h.at[src_v], a_src, sem1),
        pltpu.async_copy(assoc_h.at[dst_v], a_dst, sem1),
        pltpu.async_copy(assoc_h.at[neg_v], a_neg, sem1),
    ]
    for r in range(S):
        hop1.append(pltpu.async_copy(assoc_h.at[nds_v.at[r]], a_nds.at[r], sem1))
        hop1.append(pltpu.async_copy(assoc_h.at[nss_v.at[r]], a_nss.at[r], sem1))
    for c in hop1:
        c.wait()

    for t in range(ECH):
        sl = pl.ds(t * L, L)
        for a_ref, i1_ref in ((a_src, i1s), (a_dst, i1d), (a_neg, i1n)):
            i1_ref[sl] = a_ref[sl] + NPAD

    def idx_body(r, carry):
        for t in range(ECH):
            sl = pl.ds(t * L, L)
            for a_ref, i1_ref in ((a_nds, i1k), (a_nss, i1m)):
                i1_ref[r, sl] = a_ref[r, sl] + NPAD
        return carry

    lax.fori_loop(0, S, idx_body, 0)

    hop2 = []
    for a_ref, i1_ref, outs in (
            (a_src, i1s, (s0u, s1u, luu)),
            (a_dst, i1d, (s0v, s1v, luv)),
            (a_neg, i1n, (s0w, s1w, luw))):
        hop2.append(pltpu.async_copy(stab_h.at[a_ref], outs[0], sem2))
        hop2.append(pltpu.async_copy(stab_h.at[i1_ref], outs[1], sem2))
        hop2.append(pltpu.async_copy(lu_h.at[a_ref], outs[2], sem2))
    for a_ref, i1_ref, outs in (
            (a_nds, i1k, (s0k, s1k, luk)),
            (a_nss, i1m, (s0m, s1m, lum))):
        for r in range(S):
            hop2.append(pltpu.async_copy(stab_h.at[a_ref.at[r]], outs[0].at[r], sem2))
            hop2.append(pltpu.async_copy(stab_h.at[i1_ref.at[r]], outs[1].at[r], sem2))
            hop2.append(pltpu.async_copy(lu_h.at[a_ref.at[r]], outs[2].at[r], sem2))
    for c in hop2:
        c.wait()

    psi0 = par_v[pl.ds(0 * L, L)]
    psi1 = par_v[pl.ds(1 * L, L)]
    al0 = par_v[pl.ds(2 * L, L)]
    al1 = par_v[pl.ds(3 * L, L)]
    wt0 = par_v[pl.ds(4 * L, L)]
    wt1 = par_v[pl.ds(5 * L, L)]
    b0v = par_v[pl.ds(6 * L, L)]
    b1v = par_v[pl.ds(7 * L, L)]

    def chunk_body(c, carry):
        sl = pl.ds(c * L, L)
        etm = et_v[sl] > 0
        ctc = ct_v[sl]
        lts = luu[sl]
        ltd = luv[sl]
        s0uc = s0u[sl]
        s1uc = s1u[sl]
        s0dc = s0v[sl]
        s1dc = s1v[sl]

        psie = jnp.where(etm, psi1, psi0)
        ale = jnp.where(etm, al1, al0)
        wte = jnp.where(etm, wt1, wt0)
        be = jnp.where(etm, b1v, b0v)

        ssum = jnp.where(etm, s1uc + s1dc, s0uc + s0dc)
        lam_v[sl] = _hawkes(ssum, psie, ale, wte, be,
                            ctc - jnp.maximum(lts, ltd))

        ssumn = jnp.where(etm, s1uc + s1w[sl], s0uc + s0w[sl])
        lamn_v[sl] = _hawkes(ssumn, psie, ale, wte, be,
                             ctc - jnp.maximum(lts, luw[sl]))

        def jbody(j, accs):
            au, av = accs
            ksl = pl.ds(c * L, L)
            tdu = ctc - jnp.maximum(lts, luk[j, ksl])
            au = (au
                  + _hawkes(s0uc + s0k[j, ksl], psi0, al0, wt0, b0v, tdu)
                  + _hawkes(s1uc + s1k[j, ksl], psi1, al1, wt1, b1v, tdu))
            tdv = ctc - jnp.maximum(lum[j, ksl], ltd)
            av = (av
                  + _hawkes(s0m[j, ksl] + s0dc, psi0, al0, wt0, b0v, tdv)
                  + _hawkes(s1m[j, ksl] + s1dc, psi1, al1, wt1, b1v, tdv))
            return (au, av)

        zero = jnp.zeros((L,), jnp.float32)
        acc_u, acc_v = lax.fori_loop(0, S, jbody, (zero, zero))
        susum_v[sl] = acc_u
        svsum_v[sl] = acc_v
        return carry

    lax.fori_loop(0, ECH, chunk_body, 0)

    pltpu.sync_copy(lam_v, lam_o.at[pl.ds(eb, EPW)])
    pltpu.sync_copy(lamn_v, lamn_o.at[pl.ds(eb, EPW)])
    pltpu.sync_copy(susum_v, susum_o.at[pl.ds(eb, EPW)])
    pltpu.sync_copy(svsum_v, svsum_o.at[pl.ds(eb, EPW)])


_sc_call = pl.kernel(
    _sc_body,
    out_type=[jax.ShapeDtypeStruct((B,), jnp.float32)] * 4,
    mesh=plsc.VectorSubcoreMesh(core_axis_name="c", subcore_axis_name="s",
                                num_cores=NC, num_subcores=NS),
    scratch_types=[
        pltpu.VMEM((EPW,), jnp.int32),
        pltpu.VMEM((EPW,), jnp.int32),
        pltpu.VMEM((EPW,), jnp.int32),
        pltpu.VMEM((EPW,), jnp.int32),
        pltpu.VMEM((EPW,), jnp.float32),
        pltpu.VMEM((S, EPW), jnp.int32),
        pltpu.VMEM((S, EPW), jnp.int32),
        pltpu.VMEM((S, EPW), jnp.int32),
        pltpu.VMEM((EPW,), jnp.int32),
        pltpu.VMEM((EPW,), jnp.int32),
        pltpu.VMEM((EPW,), jnp.int32),
        pltpu.VMEM((S, EPW), jnp.int32),
        pltpu.VMEM((S, EPW), jnp.int32),
        pltpu.VMEM((EPW,), jnp.int32),
        pltpu.VMEM((EPW,), jnp.int32),
        pltpu.VMEM((EPW,), jnp.int32),
        pltpu.VMEM((S, EPW), jnp.int32),
        pltpu.VMEM((S, EPW), jnp.int32),
        pltpu.VMEM((EPW,), jnp.float32),
        pltpu.VMEM((EPW,), jnp.float32),
        pltpu.VMEM((EPW,), jnp.float32),
        pltpu.VMEM((EPW,), jnp.float32),
        pltpu.VMEM((EPW,), jnp.float32),
        pltpu.VMEM((EPW,), jnp.float32),
        pltpu.VMEM((EPW,), jnp.float32),
        pltpu.VMEM((EPW,), jnp.float32),
        pltpu.VMEM((EPW,), jnp.float32),
        pltpu.VMEM((S, EPW), jnp.float32),
        pltpu.VMEM((S, EPW), jnp.float32),
        pltpu.VMEM((S, EPW), jnp.float32),
        pltpu.VMEM((S, EPW), jnp.float32),
        pltpu.VMEM((S, EPW), jnp.float32),
        pltpu.VMEM((S, EPW), jnp.float32),
        pltpu.VMEM((8 * L,), jnp.float32),
        pltpu.VMEM((EPW,), jnp.float32),
        pltpu.VMEM((EPW,), jnp.float32),
        pltpu.VMEM((EPW,), jnp.float32),
        pltpu.VMEM((EPW,), jnp.float32),
        pltpu.SemaphoreType.DMA,
        pltpu.SemaphoreType.DMA,
        pltpu.SemaphoreType.DMA,
    ],
)


def _fin_body(lam_ref, lamn_ref, su_ref, sv_ref,
              ll_ref, lsu_ref, lsv_ref, cp_ref, cn_ref):
    lam = lam_ref[...]
    lamn = lamn_ref[...]
    su = su_ref[...]
    sv = sv_ref[...]
    ll_ref[...] = -jnp.sum(jnp.log(lam + 1e-7), keepdims=True) / B
    lsu_ref[...] = jnp.sum(su, keepdims=True) / (S * B)
    lsv_ref[...] = jnp.sum(sv, keepdims=True) / (S * B)
    surv = jnp.exp(-(su + sv) / S)
    cp_ref[...] = lam * surv
    cn_ref[...] = lamn * surv


_fin_call = pl.pallas_call(
    _fin_body,
    out_shape=[
        jax.ShapeDtypeStruct((1, 1), jnp.float32),
        jax.ShapeDtypeStruct((1, 1), jnp.float32),
        jax.ShapeDtypeStruct((1, 1), jnp.float32),
        jax.ShapeDtypeStruct((B // 128, 128), jnp.float32),
        jax.ShapeDtypeStruct((B // 128, 128), jnp.float32),
    ],
)


def kernel(all_embeddings, assoc, src, pos_dst, neg_dst_surv, neg_src_surv,
           neg_dst, last_update, cur_time, et, W0, b0, W1, b1, psi, alpha, w_t):
    ws0 = (W0[:D] + W0[D:]).astype(jnp.float32)
    ws1 = (W1[:D] + W1[D:]).astype(jnp.float32)
    w2 = jnp.zeros((8, D), jnp.float32).at[0].set(ws0).at[1].set(ws1)
    stab = _proj_call(all_embeddings, w2)

    par = jnp.concatenate([
        jnp.broadcast_to(p.astype(jnp.float32), (L,))
        for p in (psi[0], psi[1], alpha[0], alpha[1], w_t[0], w_t[1],
                  b0[0], b1[0])
    ])

    lam, lamn, susum, svsum = _sc_call(
        stab[:2].reshape(2 * NPAD), assoc.astype(jnp.int32), last_update,
        src.astype(jnp.int32), pos_dst.astype(jnp.int32),
        neg_dst.astype(jnp.int32), et.astype(jnp.int32), cur_time,
        neg_dst_surv.astype(jnp.int32), neg_src_surv.astype(jnp.int32), par)

    ll, lsu, lsv, cp, cn = _fin_call(
        lam.reshape(B // 128, 128), lamn.reshape(B // 128, 128),
        susum.reshape(B // 128, 128), svsum.reshape(B // 128, 128))
    return (ll[0, 0], lsu[0, 0], lsv[0, 0], cp.reshape(B), cn.reshape(B))


# speedup vs baseline: 9.4289x; 1.0019x over previous
import functools

import jax
import jax.numpy as jnp
from jax import lax
from jax.experimental import pallas as pl
from jax.experimental.pallas import tpu as pltpu
from jax.experimental.pallas import tpu_sc as plsc

N = 100000
B = 4096
S = 20
D = 32
TRAIN_TD_MAX = 1.0

NC = 2
NS = 16
NW = NC * NS
L = 16
EPW = B // NW
ECH = EPW // L
MPW = EPW * S
NPAD = 100096
PROJ_BLK = 50048


def _proj_body(emb_ref, w_ref, o_ref):
    o_ref[...] = lax.dot_general(w_ref[...], emb_ref[...],
                                 (((1,), (1,)), ((), ())),
                                 preferred_element_type=jnp.float32)


_proj_call = pl.pallas_call(
    _proj_body,
    grid=(NPAD // PROJ_BLK,),
    in_specs=[
        pl.BlockSpec((PROJ_BLK, D), lambda i: (i, 0)),
        pl.BlockSpec((8, D), lambda i: (0, 0)),
    ],
    out_specs=pl.BlockSpec((8, PROJ_BLK), lambda i: (0, i)),
    out_shape=jax.ShapeDtypeStruct((8, NPAD), jnp.float32),
)


def _softplus(x):
    t = jnp.exp(-jnp.abs(x))
    z = t / (2.0 + t)
    z2 = z * z
    l1p = 2.0 * z * (1.0 + z2 * (1.0 / 3.0 + z2 * (1.0 / 5.0 + z2 * (1.0 / 7.0 + z2 * (1.0 / 9.0)))))
    return jnp.maximum(x, 0.0) + l1p


def _hawkes(ssum, psi_, al_, wt_, b_, td):
    g = 0.5 * ssum + b_ + al_ * jnp.exp(-wt_ * (td / TRAIN_TD_MAX))
    x = jnp.clip(g / (psi_ + 1e-7), -75.0, 75.0)
    return psi_ * _softplus(x)


def _sc_body(stab_h, assoc_h, lu_h, src_h, dst_h, neg_h, et_h, ct_h,
             nds_h, nss_h, par_h,
             lam_o, lamn_o, susum_o, svsum_o,
             src_v, dst_v, neg_v, et_v, ct_v, rix, nds_v, nss_v,
             a_src, a_dst, a_neg, a_nds, a_nss,
             i1s, i1d, i1n, i1k, i1m,
             s0u, s1u, luu, s0v, s1v, luv, s0w, s1w, luw,
             s0k, s1k, luk, s0m, s1m, lum,
             par_v, lam_v, lamn_v, susum_v, svsum_v,
             sem0, sem1, sem2):
    wid = lax.axis_index("s") * NC + lax.axis_index("c")
    eb = wid * EPW
    mb = wid * EPW * S

    pltpu.sync_copy(src_h.at[pl.ds(eb, EPW)], src_v)
    pltpu.sync_copy(dst_h.at[pl.ds(eb, EPW)], dst_v)
    pltpu.sync_copy(neg_h.at[pl.ds(eb, EPW)], neg_v)
    pltpu.sync_copy(et_h.at[pl.ds(eb, EPW)], et_v)
    pltpu.sync_copy(ct_h.at[pl.ds(eb, EPW)], ct_v)
    pltpu.sync_copy(par_h, par_v)

    lanes = lax.iota(jnp.int32, L)
    for r in range(S):
        for t in range(ECH):
            rix[pl.ds(r * EPW + t * L, L)] = mb + (lanes + t * L) * S + r
    c0a = pltpu.async_copy(nds_h.at[rix], nds_v, sem0)
    c0b = pltpu.async_copy(nss_h.at[rix], nss_v, sem0)
    c0a.wait()
    c0b.wait()

    hop1 = [
        pltpu.async_copy(assoc_h.at[src_v], a_src, sem1),
        pltpu.async_copy(assoc_h.at[dst_v], a_dst, sem1),
        pltpu.async_copy(assoc_h.at[neg_v], a_neg, sem1),
        pltpu.async_copy(assoc_h.at[nds_v], a_nds, sem1),
        pltpu.async_copy(assoc_h.at[nss_v], a_nss, sem1),
    ]
    for c in hop1:
        c.wait()

    for t in range(ECH):
        sl = pl.ds(t * L, L)
        for a_ref, i1_ref in ((a_src, i1s), (a_dst, i1d), (a_neg, i1n)):
            i1_ref[sl] = a_ref[sl] + NPAD

    def idx_body(r, carry):
        sl = pl.ds(r * EPW, EPW)
        for t in range(ECH):
            tsl = pl.ds(r * EPW + t * L, L)
            for a_ref, i1_ref in ((a_nds, i1k), (a_nss, i1m)):
                i1_ref[tsl] = a_ref[tsl] + NPAD
        return carry

    lax.fori_loop(0, S, idx_body, 0)

    hop2 = []
    for a_ref, i1_ref, outs in (
            (a_src, i1s, (s0u, s1u, luu)),
            (a_dst, i1d, (s0v, s1v, luv)),
            (a_neg, i1n, (s0w, s1w, luw))):
        hop2.append(pltpu.async_copy(stab_h.at[a_ref], outs[0], sem2))
        hop2.append(pltpu.async_copy(stab_h.at[i1_ref], outs[1], sem2))
        hop2.append(pltpu.async_copy(lu_h.at[a_ref], outs[2], sem2))
    for a_ref, i1_ref, outs in (
            (a_nds, i1k, (s0k, s1k, luk)),
            (a_nss, i1m, (s0m, s1m, lum))):
        hop2.append(pltpu.async_copy(stab_h.at[a_ref], outs[0], sem2))
        hop2.append(pltpu.async_copy(stab_h.at[i1_ref], outs[1], sem2))
        hop2.append(pltpu.async_copy(lu_h.at[a_ref], outs[2], sem2))
    for c in hop2:
        c.wait()

    psi0 = par_v[pl.ds(0 * L, L)]
    psi1 = par_v[pl.ds(1 * L, L)]
    al0 = par_v[pl.ds(2 * L, L)]
    al1 = par_v[pl.ds(3 * L, L)]
    wt0 = par_v[pl.ds(4 * L, L)]
    wt1 = par_v[pl.ds(5 * L, L)]
    b0v = par_v[pl.ds(6 * L, L)]
    b1v = par_v[pl.ds(7 * L, L)]

    def chunk_body(c, carry):
        sl = pl.ds(c * L, L)
        etm = et_v[sl] > 0
        ctc = ct_v[sl]
        lts = luu[sl]
        ltd = luv[sl]
        s0uc = s0u[sl]
        s1uc = s1u[sl]
        s0dc = s0v[sl]
        s1dc = s1v[sl]

        psie = jnp.where(etm, psi1, psi0)
        ale = jnp.where(etm, al1, al0)
        wte = jnp.where(etm, wt1, wt0)
        be = jnp.where(etm, b1v, b0v)

        ssum = jnp.where(etm, s1uc + s1dc, s0uc + s0dc)
        lam_v[sl] = _hawkes(ssum, psie, ale, wte, be,
                            ctc - jnp.maximum(lts, ltd))

        ssumn = jnp.where(etm, s1uc + s1w[sl], s0uc + s0w[sl])
        lamn_v[sl] = _hawkes(ssumn, psie, ale, wte, be,
                             ctc - jnp.maximum(lts, luw[sl]))

        def jbody(j, accs):
            au, av = accs
            ksl = pl.ds(j * EPW + c * L, L)
            tdu = ctc - jnp.maximum(lts, luk[ksl])
            au = (au
                  + _hawkes(s0uc + s0k[ksl], psi0, al0, wt0, b0v, tdu)
                  + _hawkes(s1uc + s1k[ksl], psi1, al1, wt1, b1v, tdu))
            tdv = ctc - jnp.maximum(lum[ksl], ltd)
            av = (av
                  + _hawkes(s0m[ksl] + s0dc, psi0, al0, wt0, b0v, tdv)
                  + _hawkes(s1m[ksl] + s1dc, psi1, al1, wt1, b1v, tdv))
            return (au, av)

        zero = jnp.zeros((L,), jnp.float32)
        acc_u, acc_v = lax.fori_loop(0, S, jbody, (zero, zero))
        susum_v[sl] = acc_u
        svsum_v[sl] = acc_v
        return carry

    lax.fori_loop(0, ECH, chunk_body, 0)

    pltpu.sync_copy(lam_v, lam_o.at[pl.ds(eb, EPW)])
    pltpu.sync_copy(lamn_v, lamn_o.at[pl.ds(eb, EPW)])
    pltpu.sync_copy(susum_v, susum_o.at[pl.ds(eb, EPW)])
    pltpu.sync_copy(svsum_v, svsum_o.at[pl.ds(eb, EPW)])


_sc_call = pl.kernel(
    _sc_body,
    out_type=[jax.ShapeDtypeStruct((B,), jnp.float32)] * 4,
    mesh=plsc.VectorSubcoreMesh(core_axis_name="c", subcore_axis_name="s",
                                num_cores=NC, num_subcores=NS),
    scratch_types=[
        pltpu.VMEM((EPW,), jnp.int32),
        pltpu.VMEM((EPW,), jnp.int32),
        pltpu.VMEM((EPW,), jnp.int32),
        pltpu.VMEM((EPW,), jnp.int32),
        pltpu.VMEM((EPW,), jnp.float32),
        pltpu.VMEM((MPW,), jnp.int32),
        pltpu.VMEM((MPW,), jnp.int32),
        pltpu.VMEM((MPW,), jnp.int32),
        pltpu.VMEM((EPW,), jnp.int32),
        pltpu.VMEM((EPW,), jnp.int32),
        pltpu.VMEM((EPW,), jnp.int32),
        pltpu.VMEM((MPW,), jnp.int32),
        pltpu.VMEM((MPW,), jnp.int32),
        pltpu.VMEM((EPW,), jnp.int32),
        pltpu.VMEM((EPW,), jnp.int32),
        pltpu.VMEM((EPW,), jnp.int32),
        pltpu.VMEM((MPW,), jnp.int32),
        pltpu.VMEM((MPW,), jnp.int32),
        pltpu.VMEM((EPW,), jnp.float32),
        pltpu.VMEM((EPW,), jnp.float32),
        pltpu.VMEM((EPW,), jnp.float32),
        pltpu.VMEM((EPW,), jnp.float32),
        pltpu.VMEM((EPW,), jnp.float32),
        pltpu.VMEM((EPW,), jnp.float32),
        pltpu.VMEM((EPW,), jnp.float32),
        pltpu.VMEM((EPW,), jnp.float32),
        pltpu.VMEM((EPW,), jnp.float32),
        pltpu.VMEM((MPW,), jnp.float32),
        pltpu.VMEM((MPW,), jnp.float32),
        pltpu.VMEM((MPW,), jnp.float32),
        pltpu.VMEM((MPW,), jnp.float32),
        pltpu.VMEM((MPW,), jnp.float32),
        pltpu.VMEM((MPW,), jnp.float32),
        pltpu.VMEM((8 * L,), jnp.float32),
        pltpu.VMEM((EPW,), jnp.float32),
        pltpu.VMEM((EPW,), jnp.float32),
        pltpu.VMEM((EPW,), jnp.float32),
        pltpu.VMEM((EPW,), jnp.float32),
        pltpu.SemaphoreType.DMA,
        pltpu.SemaphoreType.DMA,
        pltpu.SemaphoreType.DMA,
    ],
)


def _fin_body(lam_ref, lamn_ref, su_ref, sv_ref,
              ll_ref, lsu_ref, lsv_ref, cp_ref, cn_ref):
    lam = lam_ref[...]
    lamn = lamn_ref[...]
    su = su_ref[...]
    sv = sv_ref[...]
    ll_ref[...] = -jnp.sum(jnp.log(lam + 1e-7), keepdims=True) / B
    lsu_ref[...] = jnp.sum(su, keepdims=True) / (S * B)
    lsv_ref[...] = jnp.sum(sv, keepdims=True) / (S * B)
    surv = jnp.exp(-(su + sv) / S)
    cp_ref[...] = lam * surv
    cn_ref[...] = lamn * surv


_fin_call = pl.pallas_call(
    _fin_body,
    out_shape=[
        jax.ShapeDtypeStruct((1, 1), jnp.float32),
        jax.ShapeDtypeStruct((1, 1), jnp.float32),
        jax.ShapeDtypeStruct((1, 1), jnp.float32),
        jax.ShapeDtypeStruct((B // 128, 128), jnp.float32),
        jax.ShapeDtypeStruct((B // 128, 128), jnp.float32),
    ],
)


def kernel(all_embeddings, assoc, src, pos_dst, neg_dst_surv, neg_src_surv,
           neg_dst, last_update, cur_time, et, W0, b0, W1, b1, psi, alpha, w_t):
    ws0 = (W0[:D] + W0[D:]).astype(jnp.float32)
    ws1 = (W1[:D] + W1[D:]).astype(jnp.float32)
    w2 = jnp.zeros((8, D), jnp.float32).at[0].set(ws0).at[1].set(ws1)
    stab = _proj_call(all_embeddings, w2)

    par = jnp.concatenate([
        jnp.broadcast_to(p.astype(jnp.float32), (L,))
        for p in (psi[0], psi[1], alpha[0], alpha[1], w_t[0], w_t[1],
                  b0[0], b1[0])
    ])

    lam, lamn, susum, svsum = _sc_call(
        stab.reshape(8 * NPAD), assoc.astype(jnp.int32), last_update,
        src.astype(jnp.int32), pos_dst.astype(jnp.int32),
        neg_dst.astype(jnp.int32), et.astype(jnp.int32), cur_time,
        neg_dst_surv.astype(jnp.int32), neg_src_surv.astype(jnp.int32), par)

    ll, lsu, lsv, cp, cn = _fin_call(
        lam.reshape(B // 128, 128), lamn.reshape(B // 128, 128),
        susum.reshape(B // 128, 128), svsum.reshape(B // 128, 128))
    return (ll[0, 0], lsu[0, 0], lsv[0, 0], cp.reshape(B), cn.reshape(B))


# speedup vs baseline: 14.4018x; 1.5274x over previous
import functools

import jax
import jax.numpy as jnp
from jax import lax
from jax.experimental import pallas as pl
from jax.experimental.pallas import tpu as pltpu
from jax.experimental.pallas import tpu_sc as plsc

N = 100000
B = 4096
S = 20
D = 32
TRAIN_TD_MAX = 1.0

NC = 2
NS = 16
NW = NC * NS
L = 16
EPW = B // NW
ECH = EPW // L
MPW = EPW * S
NPAD = 100096
PROJ_BLK = 50048


def _proj_body(embt_ref, w_ref, o_ref):
    o_ref[...] = lax.dot_general(w_ref[...], embt_ref[...],
                                 (((1,), (0,)), ((), ())),
                                 preferred_element_type=jnp.float32)


_proj_call = pl.pallas_call(
    _proj_body,
    grid=(NPAD // PROJ_BLK,),
    in_specs=[
        pl.BlockSpec((D, PROJ_BLK), lambda i: (0, i)),
        pl.BlockSpec((8, D), lambda i: (0, 0)),
    ],
    out_specs=pl.BlockSpec((8, PROJ_BLK), lambda i: (0, i)),
    out_shape=jax.ShapeDtypeStruct((8, NPAD), jnp.float32),
)


def _softplus(x):
    t = jnp.exp(-jnp.abs(x))
    z = t / (2.0 + t)
    z2 = z * z
    l1p = 2.0 * z * (1.0 + z2 * (1.0 / 3.0 + z2 * (1.0 / 5.0 + z2 * (1.0 / 7.0 + z2 * (1.0 / 9.0)))))
    return jnp.maximum(x, 0.0) + l1p


def _hawkes(ssum, psi_, al_, wt_, b_, td):
    g = 0.5 * ssum + b_ + al_ * jnp.exp(-wt_ * (td / TRAIN_TD_MAX))
    x = jnp.clip(g / (psi_ + 1e-7), -75.0, 75.0)
    return psi_ * _softplus(x)


def _sc_body(stab_h, assoc_h, lu_h, src_h, dst_h, neg_h, et_h, ct_h,
             nds_h, nss_h, par_h,
             lam_o, lamn_o, susum_o, svsum_o,
             src_v, dst_v, neg_v, et_v, ct_v, rix, nds_v, nss_v,
             a_src, a_dst, a_neg, a_nds, a_nss,
             i1s, i1d, i1n, i1k, i1m,
             s0u, s1u, luu, s0v, s1v, luv, s0w, s1w, luw,
             s0k, s1k, luk, s0m, s1m, lum,
             par_v, lam_v, lamn_v, susum_v, svsum_v,
             sem0, sem1, sem2):
    wid = lax.axis_index("s") * NC + lax.axis_index("c")
    eb = wid * EPW
    mb = wid * EPW * S

    pltpu.sync_copy(src_h.at[pl.ds(eb, EPW)], src_v)
    pltpu.sync_copy(dst_h.at[pl.ds(eb, EPW)], dst_v)
    pltpu.sync_copy(neg_h.at[pl.ds(eb, EPW)], neg_v)
    pltpu.sync_copy(et_h.at[pl.ds(eb, EPW)], et_v)
    pltpu.sync_copy(ct_h.at[pl.ds(eb, EPW)], ct_v)
    pltpu.sync_copy(par_h, par_v)

    lanes = lax.iota(jnp.int32, L)
    for r in range(S):
        for t in range(ECH):
            rix[pl.ds(r * EPW + t * L, L)] = mb + (lanes + t * L) * S + r
    c0a = pltpu.async_copy(nds_h.at[rix], nds_v, sem0)
    c0b = pltpu.async_copy(nss_h.at[rix], nss_v, sem0)
    c0a.wait()
    c0b.wait()

    hop1 = [
        pltpu.async_copy(assoc_h.at[src_v], a_src, sem1),
        pltpu.async_copy(assoc_h.at[dst_v], a_dst, sem1),
        pltpu.async_copy(assoc_h.at[neg_v], a_neg, sem1),
        pltpu.async_copy(assoc_h.at[nds_v], a_nds, sem1),
        pltpu.async_copy(assoc_h.at[nss_v], a_nss, sem1),
    ]
    for c in hop1:
        c.wait()

    for t in range(ECH):
        sl = pl.ds(t * L, L)
        for a_ref, i1_ref in ((a_src, i1s), (a_dst, i1d), (a_neg, i1n)):
            i1_ref[sl] = a_ref[sl] + NPAD

    def idx_body(r, carry):
        sl = pl.ds(r * EPW, EPW)
        for t in range(ECH):
            tsl = pl.ds(r * EPW + t * L, L)
            for a_ref, i1_ref in ((a_nds, i1k), (a_nss, i1m)):
                i1_ref[tsl] = a_ref[tsl] + NPAD
        return carry

    lax.fori_loop(0, S, idx_body, 0)

    hop2 = []
    for a_ref, i1_ref, outs in (
            (a_src, i1s, (s0u, s1u, luu)),
            (a_dst, i1d, (s0v, s1v, luv)),
            (a_neg, i1n, (s0w, s1w, luw))):
        hop2.append(pltpu.async_copy(stab_h.at[a_ref], outs[0], sem2))
        hop2.append(pltpu.async_copy(stab_h.at[i1_ref], outs[1], sem2))
        hop2.append(pltpu.async_copy(lu_h.at[a_ref], outs[2], sem2))
    for a_ref, i1_ref, outs in (
            (a_nds, i1k, (s0k, s1k, luk)),
            (a_nss, i1m, (s0m, s1m, lum))):
        hop2.append(pltpu.async_copy(stab_h.at[a_ref], outs[0], sem2))
        hop2.append(pltpu.async_copy(stab_h.at[i1_ref], outs[1], sem2))
        hop2.append(pltpu.async_copy(lu_h.at[a_ref], outs[2], sem2))
    for c in hop2:
        c.wait()

    psi0 = par_v[pl.ds(0 * L, L)]
    psi1 = par_v[pl.ds(1 * L, L)]
    al0 = par_v[pl.ds(2 * L, L)]
    al1 = par_v[pl.ds(3 * L, L)]
    wt0 = par_v[pl.ds(4 * L, L)]
    wt1 = par_v[pl.ds(5 * L, L)]
    b0v = par_v[pl.ds(6 * L, L)]
    b1v = par_v[pl.ds(7 * L, L)]

    def chunk_body(c, carry):
        sl = pl.ds(c * L, L)
        etm = et_v[sl] > 0
        ctc = ct_v[sl]
        lts = luu[sl]
        ltd = luv[sl]
        s0uc = s0u[sl]
        s1uc = s1u[sl]
        s0dc = s0v[sl]
        s1dc = s1v[sl]

        psie = jnp.where(etm, psi1, psi0)
        ale = jnp.where(etm, al1, al0)
        wte = jnp.where(etm, wt1, wt0)
        be = jnp.where(etm, b1v, b0v)

        ssum = jnp.where(etm, s1uc + s1dc, s0uc + s0dc)
        lam_v[sl] = _hawkes(ssum, psie, ale, wte, be,
                            ctc - jnp.maximum(lts, ltd))

        ssumn = jnp.where(etm, s1uc + s1w[sl], s0uc + s0w[sl])
        lamn_v[sl] = _hawkes(ssumn, psie, ale, wte, be,
                             ctc - jnp.maximum(lts, luw[sl]))

        def jbody(j, accs):
            au, av = accs
            ksl = pl.ds(j * EPW + c * L, L)
            tdu = ctc - jnp.maximum(lts, luk[ksl])
            au = (au
                  + _hawkes(s0uc + s0k[ksl], psi0, al0, wt0, b0v, tdu)
                  + _hawkes(s1uc + s1k[ksl], psi1, al1, wt1, b1v, tdu))
            tdv = ctc - jnp.maximum(lum[ksl], ltd)
            av = (av
                  + _hawkes(s0m[ksl] + s0dc, psi0, al0, wt0, b0v, tdv)
                  + _hawkes(s1m[ksl] + s1dc, psi1, al1, wt1, b1v, tdv))
            return (au, av)

        zero = jnp.zeros((L,), jnp.float32)
        acc_u, acc_v = lax.fori_loop(0, S, jbody, (zero, zero))
        susum_v[sl] = acc_u
        svsum_v[sl] = acc_v
        return carry

    lax.fori_loop(0, ECH, chunk_body, 0)

    pltpu.sync_copy(lam_v, lam_o.at[pl.ds(eb, EPW)])
    pltpu.sync_copy(lamn_v, lamn_o.at[pl.ds(eb, EPW)])
    pltpu.sync_copy(susum_v, susum_o.at[pl.ds(eb, EPW)])
    pltpu.sync_copy(svsum_v, svsum_o.at[pl.ds(eb, EPW)])


_sc_call = pl.kernel(
    _sc_body,
    out_type=[jax.ShapeDtypeStruct((B,), jnp.float32)] * 4,
    mesh=plsc.VectorSubcoreMesh(core_axis_name="c", subcore_axis_name="s",
                                num_cores=NC, num_subcores=NS),
    scratch_types=[
        pltpu.VMEM((EPW,), jnp.int32),
        pltpu.VMEM((EPW,), jnp.int32),
        pltpu.VMEM((EPW,), jnp.int32),
        pltpu.VMEM((EPW,), jnp.int32),
        pltpu.VMEM((EPW,), jnp.float32),
        pltpu.VMEM((MPW,), jnp.int32),
        pltpu.VMEM((MPW,), jnp.int32),
        pltpu.VMEM((MPW,), jnp.int32),
        pltpu.VMEM((EPW,), jnp.int32),
        pltpu.VMEM((EPW,), jnp.int32),
        pltpu.VMEM((EPW,), jnp.int32),
        pltpu.VMEM((MPW,), jnp.int32),
        pltpu.VMEM((MPW,), jnp.int32),
        pltpu.VMEM((EPW,), jnp.int32),
        pltpu.VMEM((EPW,), jnp.int32),
        pltpu.VMEM((EPW,), jnp.int32),
        pltpu.VMEM((MPW,), jnp.int32),
        pltpu.VMEM((MPW,), jnp.int32),
        pltpu.VMEM((EPW,), jnp.float32),
        pltpu.VMEM((EPW,), jnp.float32),
        pltpu.VMEM((EPW,), jnp.float32),
        pltpu.VMEM((EPW,), jnp.float32),
        pltpu.VMEM((EPW,), jnp.float32),
        pltpu.VMEM((EPW,), jnp.float32),
        pltpu.VMEM((EPW,), jnp.float32),
        pltpu.VMEM((EPW,), jnp.float32),
        pltpu.VMEM((EPW,), jnp.float32),
        pltpu.VMEM((MPW,), jnp.float32),
        pltpu.VMEM((MPW,), jnp.float32),
        pltpu.VMEM((MPW,), jnp.float32),
        pltpu.VMEM((MPW,), jnp.float32),
        pltpu.VMEM((MPW,), jnp.float32),
        pltpu.VMEM((MPW,), jnp.float32),
        pltpu.VMEM((8 * L,), jnp.float32),
        pltpu.VMEM((EPW,), jnp.float32),
        pltpu.VMEM((EPW,), jnp.float32),
        pltpu.VMEM((EPW,), jnp.float32),
        pltpu.VMEM((EPW,), jnp.float32),
        pltpu.SemaphoreType.DMA,
        pltpu.SemaphoreType.DMA,
        pltpu.SemaphoreType.DMA,
    ],
)


def _fin_body(lam_ref, lamn_ref, su_ref, sv_ref,
              ll_ref, lsu_ref, lsv_ref, cp_ref, cn_ref):
    lam = lam_ref[...]
    lamn = lamn_ref[...]
    su = su_ref[...]
    sv = sv_ref[...]
    ll_ref[...] = -jnp.sum(jnp.log(lam + 1e-7), keepdims=True) / B
    lsu_ref[...] = jnp.sum(su, keepdims=True) / (S * B)
    lsv_ref[...] = jnp.sum(sv, keepdims=True) / (S * B)
    surv = jnp.exp(-(su + sv) / S)
    cp_ref[...] = lam * surv
    cn_ref[...] = lamn * surv


_fin_call = pl.pallas_call(
    _fin_body,
    out_shape=[
        jax.ShapeDtypeStruct((1, 1), jnp.float32),
        jax.ShapeDtypeStruct((1, 1), jnp.float32),
        jax.ShapeDtypeStruct((1, 1), jnp.float32),
        jax.ShapeDtypeStruct((B // 128, 128), jnp.float32),
        jax.ShapeDtypeStruct((B // 128, 128), jnp.float32),
    ],
)


def kernel(all_embeddings, assoc, src, pos_dst, neg_dst_surv, neg_src_surv,
           neg_dst, last_update, cur_time, et, W0, b0, W1, b1, psi, alpha, w_t):
    ws0 = (W0[:D] + W0[D:]).astype(jnp.float32)
    ws1 = (W1[:D] + W1[D:]).astype(jnp.float32)
    w2 = jnp.zeros((8, D), jnp.float32).at[0].set(ws0).at[1].set(ws1)
    stab = _proj_call(all_embeddings.T, w2)

    par = jnp.concatenate([
        jnp.broadcast_to(p.astype(jnp.float32), (L,))
        for p in (psi[0], psi[1], alpha[0], alpha[1], w_t[0], w_t[1],
                  b0[0], b1[0])
    ])

    lam, lamn, susum, svsum = _sc_call(
        stab.reshape(8 * NPAD), assoc.astype(jnp.int32), last_update,
        src.astype(jnp.int32), pos_dst.astype(jnp.int32),
        neg_dst.astype(jnp.int32), et.astype(jnp.int32), cur_time,
        neg_dst_surv.astype(jnp.int32), neg_src_surv.astype(jnp.int32), par)

    ll, lsu, lsv, cp, cn = _fin_call(
        lam.reshape(B // 128, 128), lamn.reshape(B // 128, 128),
        susum.reshape(B // 128, 128), svsum.reshape(B // 128, 128))
    return (ll[0, 0], lsu[0, 0], lsv[0, 0], cp.reshape(B), cn.reshape(B))


# speedup vs baseline: 15.8539x; 1.1008x over previous
import functools

import jax
import jax.numpy as jnp
from jax import lax
from jax.experimental import pallas as pl
from jax.experimental.pallas import tpu as pltpu
from jax.experimental.pallas import tpu_sc as plsc

N = 100000
B = 4096
S = 20
D = 32
TRAIN_TD_MAX = 1.0

NC = 2
NS = 16
NW = NC * NS
L = 16
EPW = B // NW
ECH = EPW // L
MPW = EPW * S
NPAD = 100352
PROJ_BLK = 50176


def _proj_body(embt_ref, w_ref, o0_ref, o1_ref):
    o = lax.dot_general(w_ref[...], embt_ref[...],
                        (((1,), (0,)), ((), ())),
                        preferred_element_type=jnp.float32)
    o0_ref[...] = o[0]
    o1_ref[...] = o[1]


_proj_call = pl.pallas_call(
    _proj_body,
    grid=(NPAD // PROJ_BLK,),
    in_specs=[
        pl.BlockSpec((D, PROJ_BLK), lambda i: (0, i)),
        pl.BlockSpec((8, D), lambda i: (0, 0)),
    ],
    out_specs=[
        pl.BlockSpec((PROJ_BLK,), lambda i: (i,)),
        pl.BlockSpec((PROJ_BLK,), lambda i: (i,)),
    ],
    out_shape=[
        jax.ShapeDtypeStruct((NPAD,), jnp.float32),
        jax.ShapeDtypeStruct((NPAD,), jnp.float32),
    ],
)


def _softplus(x):
    t = jnp.exp(-jnp.abs(x))
    z = t / (2.0 + t)
    z2 = z * z
    l1p = 2.0 * z * (1.0 + z2 * (1.0 / 3.0 + z2 * (1.0 / 5.0 + z2 * (1.0 / 7.0 + z2 * (1.0 / 9.0)))))
    return jnp.maximum(x, 0.0) + l1p


def _hawkes(ssum, psi_, ipsi_, al_, wt_, b_, td):
    g = 0.5 * ssum + b_ + al_ * jnp.exp(-wt_ * (td / TRAIN_TD_MAX))
    x = jnp.clip(g * ipsi_, -75.0, 75.0)
    return psi_ * _softplus(x)


def _sc_body(s0_h, s1_h, assoc_h, lu_h, src_h, dst_h, neg_h, et_h, ct_h,
             nds_h, nss_h, par_h,
             lam_o, lamn_o, susum_o, svsum_o,
             src_v, dst_v, neg_v, et_v, ct_v, rix, nds_v, nss_v,
             a_src, a_dst, a_neg, a_nds, a_nss,
             s0u, s1u, luu, s0v, s1v, luv, s0w, s1w, luw,
             s0k, s1k, luk, s0m, s1m, lum,
             par_v, lam_v, lamn_v, susum_v, svsum_v,
             sem0, sem1, sem2):
    wid = lax.axis_index("s") * NC + lax.axis_index("c")
    eb = wid * EPW
    mb = wid * EPW * S

    pltpu.sync_copy(src_h.at[pl.ds(eb, EPW)], src_v)
    pltpu.sync_copy(dst_h.at[pl.ds(eb, EPW)], dst_v)
    pltpu.sync_copy(neg_h.at[pl.ds(eb, EPW)], neg_v)
    pltpu.sync_copy(et_h.at[pl.ds(eb, EPW)], et_v)
    pltpu.sync_copy(ct_h.at[pl.ds(eb, EPW)], ct_v)
    pltpu.sync_copy(par_h, par_v)

    lanes = lax.iota(jnp.int32, L)
    for r in range(S):
        for t in range(ECH):
            rix[pl.ds(r * EPW + t * L, L)] = mb + (lanes + t * L) * S + r
    c0a = pltpu.async_copy(nds_h.at[rix], nds_v, sem0)
    c0b = pltpu.async_copy(nss_h.at[rix], nss_v, sem0)
    c0a.wait()
    c0b.wait()

    hop1 = [
        pltpu.async_copy(assoc_h.at[src_v], a_src, sem1),
        pltpu.async_copy(assoc_h.at[dst_v], a_dst, sem1),
        pltpu.async_copy(assoc_h.at[neg_v], a_neg, sem1),
        pltpu.async_copy(assoc_h.at[nds_v], a_nds, sem1),
        pltpu.async_copy(assoc_h.at[nss_v], a_nss, sem1),
    ]
    for c in hop1:
        c.wait()

    hop2 = []
    for a_ref, outs in (
            (a_src, (s0u, s1u, luu)),
            (a_dst, (s0v, s1v, luv)),
            (a_neg, (s0w, s1w, luw)),
            (a_nds, (s0k, s1k, luk)),
            (a_nss, (s0m, s1m, lum))):
        hop2.append(pltpu.async_copy(s0_h.at[a_ref], outs[0], sem2))
        hop2.append(pltpu.async_copy(s1_h.at[a_ref], outs[1], sem2))
        hop2.append(pltpu.async_copy(lu_h.at[a_ref], outs[2], sem2))
    for c in hop2:
        c.wait()

    def splat(i):
        return par_v[pl.ds(i * L, L)]

    psi0, psi1 = splat(0), splat(1)
    ip0, ip1 = splat(2), splat(3)
    al0, al1 = splat(4), splat(5)
    wt0, wt1 = splat(6), splat(7)
    b0v, b1v = splat(8), splat(9)

    def chunk_body(c, carry):
        sl = pl.ds(c * L, L)
        etm = et_v[sl] > 0
        ctc = ct_v[sl]
        lts = luu[sl]
        ltd = luv[sl]
        s0uc = s0u[sl]
        s1uc = s1u[sl]
        s0dc = s0v[sl]
        s1dc = s1v[sl]

        psie = jnp.where(etm, psi1, psi0)
        ipe = jnp.where(etm, ip1, ip0)
        ale = jnp.where(etm, al1, al0)
        wte = jnp.where(etm, wt1, wt0)
        be = jnp.where(etm, b1v, b0v)

        ssum = jnp.where(etm, s1uc + s1dc, s0uc + s0dc)
        lam_v[sl] = _hawkes(ssum, psie, ipe, ale, wte, be,
                            ctc - jnp.maximum(lts, ltd))

        ssumn = jnp.where(etm, s1uc + s1w[sl], s0uc + s0w[sl])
        lamn_v[sl] = _hawkes(ssumn, psie, ipe, ale, wte, be,
                             ctc - jnp.maximum(lts, luw[sl]))

        def jbody(j, accs):
            au, av = accs
            ksl = pl.ds(j * EPW + c * L, L)
            tdu = ctc - jnp.maximum(lts, luk[ksl])
            au = (au
                  + _hawkes(s0uc + s0k[ksl], psi0, ip0, al0, wt0, b0v, tdu)
                  + _hawkes(s1uc + s1k[ksl], psi1, ip1, al1, wt1, b1v, tdu))
            tdv = ctc - jnp.maximum(lum[ksl], ltd)
            av = (av
                  + _hawkes(s0m[ksl] + s0dc, psi0, ip0, al0, wt0, b0v, tdv)
                  + _hawkes(s1m[ksl] + s1dc, psi1, ip1, al1, wt1, b1v, tdv))
            return (au, av)

        zero = jnp.zeros((L,), jnp.float32)
        acc_u, acc_v = lax.fori_loop(0, S, jbody, (zero, zero))
        susum_v[sl] = acc_u
        svsum_v[sl] = acc_v
        return carry

    lax.fori_loop(0, ECH, chunk_body, 0)

    pltpu.sync_copy(lam_v, lam_o.at[pl.ds(eb, EPW)])
    pltpu.sync_copy(lamn_v, lamn_o.at[pl.ds(eb, EPW)])
    pltpu.sync_copy(susum_v, susum_o.at[pl.ds(eb, EPW)])
    pltpu.sync_copy(svsum_v, svsum_o.at[pl.ds(eb, EPW)])


_sc_call = pl.kernel(
    _sc_body,
    out_type=[jax.ShapeDtypeStruct((B,), jnp.float32)] * 4,
    mesh=plsc.VectorSubcoreMesh(core_axis_name="c", subcore_axis_name="s",
                                num_cores=NC, num_subcores=NS),
    scratch_types=[
        pltpu.VMEM((EPW,), jnp.int32),
        pltpu.VMEM((EPW,), jnp.int32),
        pltpu.VMEM((EPW,), jnp.int32),
        pltpu.VMEM((EPW,), jnp.int32),
        pltpu.VMEM((EPW,), jnp.float32),
        pltpu.VMEM((MPW,), jnp.int32),
        pltpu.VMEM((MPW,), jnp.int32),
        pltpu.VMEM((MPW,), jnp.int32),
        pltpu.VMEM((EPW,), jnp.int32),
        pltpu.VMEM((EPW,), jnp.int32),
        pltpu.VMEM((EPW,), jnp.int32),
        pltpu.VMEM((MPW,), jnp.int32),
        pltpu.VMEM((MPW,), jnp.int32),
        pltpu.VMEM((EPW,), jnp.float32),
        pltpu.VMEM((EPW,), jnp.float32),
        pltpu.VMEM((EPW,), jnp.float32),
        pltpu.VMEM((EPW,), jnp.float32),
        pltpu.VMEM((EPW,), jnp.float32),
        pltpu.VMEM((EPW,), jnp.float32),
        pltpu.VMEM((EPW,), jnp.float32),
        pltpu.VMEM((EPW,), jnp.float32),
        pltpu.VMEM((EPW,), jnp.float32),
        pltpu.VMEM((MPW,), jnp.float32),
        pltpu.VMEM((MPW,), jnp.float32),
        pltpu.VMEM((MPW,), jnp.float32),
        pltpu.VMEM((MPW,), jnp.float32),
        pltpu.VMEM((MPW,), jnp.float32),
        pltpu.VMEM((MPW,), jnp.float32),
        pltpu.VMEM((10 * L,), jnp.float32),
        pltpu.VMEM((EPW,), jnp.float32),
        pltpu.VMEM((EPW,), jnp.float32),
        pltpu.VMEM((EPW,), jnp.float32),
        pltpu.VMEM((EPW,), jnp.float32),
        pltpu.SemaphoreType.DMA,
        pltpu.SemaphoreType.DMA,
        pltpu.SemaphoreType.DMA,
    ],
)


def _fin_body(lam_ref, lamn_ref, su_ref, sv_ref,
              ll_ref, lsu_ref, lsv_ref, cp_ref, cn_ref):
    lam = lam_ref[...]
    lamn = lamn_ref[...]
    su = su_ref[...]
    sv = sv_ref[...]
    ll_ref[...] = -jnp.sum(jnp.log(lam + 1e-7), keepdims=True) / B
    lsu_ref[...] = jnp.sum(su, keepdims=True) / (S * B)
    lsv_ref[...] = jnp.sum(sv, keepdims=True) / (S * B)
    surv = jnp.exp(-(su + sv) / S)
    cp_ref[...] = lam * surv
    cn_ref[...] = lamn * surv


_fin_call = pl.pallas_call(
    _fin_body,
    out_shape=[
        jax.ShapeDtypeStruct((1, 1), jnp.float32),
        jax.ShapeDtypeStruct((1, 1), jnp.float32),
        jax.ShapeDtypeStruct((1, 1), jnp.float32),
        jax.ShapeDtypeStruct((B // 128, 128), jnp.float32),
        jax.ShapeDtypeStruct((B // 128, 128), jnp.float32),
    ],
)


def kernel(all_embeddings, assoc, src, pos_dst, neg_dst_surv, neg_src_surv,
           neg_dst, last_update, cur_time, et, W0, b0, W1, b1, psi, alpha, w_t):
    ws0 = (W0[:D] + W0[D:]).astype(jnp.float32)
    ws1 = (W1[:D] + W1[D:]).astype(jnp.float32)
    w2 = jnp.zeros((8, D), jnp.float32).at[0].set(ws0).at[1].set(ws1)
    _proj_out = _proj_call(all_embeddings.T, w2)

    ipsi = 1.0 / (psi + 1e-7)
    par = jnp.repeat(
        jnp.stack([psi[0], psi[1], ipsi[0], ipsi[1], alpha[0], alpha[1],
                   w_t[0], w_t[1], b0[0], b1[0]]).astype(jnp.float32), L)

    s0tab, s1tab = _proj_out
    lam, lamn, susum, svsum = _sc_call(
        s0tab, s1tab, assoc.astype(jnp.int32), last_update,
        src.astype(jnp.int32), pos_dst.astype(jnp.int32),
        neg_dst.astype(jnp.int32), et.astype(jnp.int32), cur_time,
        neg_dst_surv.astype(jnp.int32), neg_src_surv.astype(jnp.int32), par)

    ll, lsu, lsv, cp, cn = _fin_call(
        lam.reshape(B // 128, 128), lamn.reshape(B // 128, 128),
        susum.reshape(B // 128, 128), svsum.reshape(B // 128, 128))
    return (ll[0, 0], lsu[0, 0], lsv[0, 0], cp.reshape(B), cn.reshape(B))
